# baseline (speedup 1.0000x reference)
import jax
import jax.numpy as jnp
from jax import lax
from jax.experimental import pallas as pl
from jax.experimental.pallas import tpu as pltpu

VOCAB = 100000
EMBED_DIM = 64
CTX_LEN = 200
HIDDEN = 64
FLAT = CTX_LEN * EMBED_DIM

CHUNKS = [(0, 24960), (24960, 24960), (49920, 24960), (74880, 25120)]
CMAX = 25120


def _body(ctx_ref, table_ref, w1r_ref, b1_ref, w2_ref, b2_ref, out_ref,
          emb_scr, w1_scr, w2a_scr, w2b_scr, sem, w1_sem, w2a_sem, w2b_sem):
    w2_scr = [w2a_scr, w2b_scr]
    w2_sem = [w2a_sem, w2b_sem]

    def chunk_copy(k):
        off, w = CHUNKS[k]
        return pltpu.make_async_copy(
            w2_ref.at[pl.ds(off, w), :],
            w2_scr[k % 2].at[pl.ds(0, w), :],
            w2_sem[k % 2],
        )

    gathers = []
    c = pltpu.make_async_copy(
        table_ref.at[pl.ds(ctx_ref[0], 1 * CTX_LEN), :],
        emb_scr.at[pl.ds(0, CTX_LEN), :],
        sem,
    )
    c.start()
    gathers.append(c)
    w1c = pltpu.make_async_copy(w1r_ref, w1_scr, w1_sem)
    w1c.start()
    pending = [chunk_copy(0), chunk_copy(1)]
    pending[0].start()
    pending[1].start()
    for c in gathers:
        c.wait()
    w1c.wait()

    h = b1_ref[...]
    for j in range(CTX_LEN):
        h = h + lax.dot_general(
            emb_scr[pl.ds(j, 1), :],
            w1_scr[pl.ds(j * EMBED_DIM, EMBED_DIM), :],
            (((1,), (0,)), ((), ())),
            preferred_element_type=jnp.float32,
        )
    h = jnp.maximum(h, 0.0)

    for k in range(len(CHUNKS)):
        off, w = CHUNKS[k]
        pending[k].wait()
        logits = lax.dot_general(
            h, w2_scr[k % 2][pl.ds(0, w), :],
            (((1,), (1,)), ((), ())),
            preferred_element_type=jnp.float32,
        ) + b2_ref[:, pl.ds(off, w)]
        out_ref[:, pl.ds(off, w)] = logits
        if k + 2 < len(CHUNKS):
            nxt = chunk_copy(k + 2)
            nxt.start()
            pending.append(nxt)

    l = out_ref[...]
    m = jnp.max(l)
    s = jnp.sum(jnp.exp(l - m))
    out_ref[...] = l - m - jnp.log(s)


def kernel(context, emb_table, W1, b1, W2, b2):
    W1r = W1.T
    return pl.pallas_call(
        _body,
        in_specs=[
            pl.BlockSpec(memory_space=pltpu.SMEM),
            pl.BlockSpec(memory_space=pl.ANY),
            pl.BlockSpec(memory_space=pl.ANY),
            pl.BlockSpec((1, HIDDEN), lambda: (0, 0)),
            pl.BlockSpec(memory_space=pl.ANY),
            pl.BlockSpec((1, VOCAB), lambda: (0, 0)),
        ],
        out_specs=pl.BlockSpec((1, VOCAB), lambda: (0, 0)),
        out_shape=jax.ShapeDtypeStruct((1, VOCAB), jnp.float32),
        scratch_shapes=[
            pltpu.VMEM((CTX_LEN, EMBED_DIM), jnp.float32),
            pltpu.VMEM((FLAT, HIDDEN), jnp.float32),
            pltpu.VMEM((CMAX, EMBED_DIM), jnp.float32),
            pltpu.VMEM((CMAX, EMBED_DIM), jnp.float32),
            pltpu.SemaphoreType.DMA,
            pltpu.SemaphoreType.DMA,
            pltpu.SemaphoreType.DMA,
            pltpu.SemaphoreType.DMA,
        ],
    )(context, emb_table, W1r, b1.reshape(1, HIDDEN), W2,
      b2.reshape(1, VOCAB))
